# baseline (speedup 1.0000x reference)
import jax
import jax.numpy as jnp
from jax.experimental import pallas as pl

_N = 10000
_H = 128
_BLOCK = 2000


def _mlp_kernel(x_ref, w1_ref, b1_ref, w2_ref, b2_ref, o_ref):
    x = x_ref[...]
    h = (x[:, 0:1] * w1_ref[0:1, :]
         + x[:, 1:2] * w1_ref[1:2, :]
         + b1_ref[...])
    h = jnp.maximum(h, 0.0)
    o_ref[...] = (jnp.dot(h, w2_ref[...], preferred_element_type=jnp.float32)
                  + b2_ref[...])


def kernel(vv_node_features, cc_node_features, vc_node_features, cv_node_features,
           edge_index_vv_cv_1, edge_index_vv_vc_2, edge_index_cc_vc_1, edge_index_cc_cv_2,
           edge_index_vc_cc_1, edge_index_vc_vv_2, edge_index_cv_vv_1, edge_index_cv_cc_2,
           W1, b1, W2, b2):
    x = vv_node_features
    b1_2d = b1.reshape(1, _H)
    b2_2d = b2.reshape(1, _H)
    grid = (_N // _BLOCK,)
    return pl.pallas_call(
        _mlp_kernel,
        grid=grid,
        in_specs=[
            pl.BlockSpec((_BLOCK, 2), lambda i: (i, 0)),
            pl.BlockSpec((2, _H), lambda i: (0, 0)),
            pl.BlockSpec((1, _H), lambda i: (0, 0)),
            pl.BlockSpec((_H, _H), lambda i: (0, 0)),
            pl.BlockSpec((1, _H), lambda i: (0, 0)),
        ],
        out_specs=pl.BlockSpec((_BLOCK, _H), lambda i: (i, 0)),
        out_shape=jax.ShapeDtypeStruct((_N, _H), jnp.float32),
    )(x, W1, b1_2d, W2, b2_2d)
